# Initial kernel scaffold; baseline (speedup 1.0000x reference)
import jax
import jax.numpy as jnp
from jax.experimental import pallas as pl


def kernel(V, edge_index, W, b):
    raise NotImplementedError("write your pallas kernel here")


# speedup vs baseline: 9.4888x; 9.4888x over previous
import functools

import jax
import jax.numpy as jnp
from jax import lax
from jax.experimental import pallas as pl
from jax.experimental.pallas import tpu as pltpu
from jax.experimental.pallas import tpu_sc as plsc

N = 10000
E = 320000
D = 128
SLOPE = 0.01

NC, NS = 2, 16
NW = NC * NS
G = 125
EPW = E // NW
NCHUNK = EPW // G
NPAD = 10240
DEG_SLICE = NPAD // NS
ROWS_PER_TILE = NPAD // NS
RCHUNK = 64

_mesh = plsc.VectorSubcoreMesh(core_axis_name="c", subcore_axis_name="s")


@functools.partial(
    pl.kernel,
    out_type=tuple(jax.ShapeDtypeStruct((NPAD,), jnp.float32) for _ in range(4)),
    mesh=_mesh,
    scratch_types=[
        pltpu.VMEM((NCHUNK, G), jnp.int32),
        pltpu.VMEM((NCHUNK, G), jnp.int32),
        pltpu.VMEM((128,), jnp.float32),
        pltpu.VMEM((DEG_SLICE,), jnp.float32),
        pltpu.VMEM_SHARED((NPAD,), jnp.float32),
        pltpu.VMEM_SHARED((NPAD,), jnp.float32),
    ],
)
def _degrees_kernel(src_hbm, dst_hbm, d00_hbm, d01_hbm, d10_hbm, d11_hbm,
                    src_v, dst_v, ones_v, buf_v, hout_sh, hin_sh):
    c = lax.axis_index("c")
    s = lax.axis_index("s")
    w = s * NC + c

    def fill_zero(i, carry):
        buf_v[pl.ds(i * 16, 16)] = jnp.zeros((16,), jnp.float32)
        return carry

    lax.fori_loop(0, DEG_SLICE // 16, fill_zero, 0)

    def fill_one(i, carry):
        ones_v[pl.ds(i * 16, 16)] = jnp.ones((16,), jnp.float32)
        return carry

    lax.fori_loop(0, 128 // 16, fill_one, 0)

    sl = pl.ds(s * DEG_SLICE, DEG_SLICE)
    pltpu.sync_copy(buf_v, hout_sh.at[sl])
    pltpu.sync_copy(buf_v, hin_sh.at[sl])
    pltpu.sync_copy(src_hbm.at[pl.ds(w * NCHUNK, NCHUNK), :], src_v)
    pltpu.sync_copy(dst_hbm.at[pl.ds(w * NCHUNK, NCHUNK), :], dst_v)
    plsc.subcore_barrier()

    ones_sl = ones_v.at[pl.ds(0, G)]

    def body(j, carry):
        pltpu.sync_copy(ones_sl, hout_sh.at[src_v.at[j]], add=True)
        pltpu.sync_copy(ones_sl, hin_sh.at[dst_v.at[j]], add=True)
        return carry

    lax.fori_loop(0, NCHUNK, body, 0)
    plsc.subcore_barrier()

    @pl.when(c == 0)
    def _():
        pltpu.sync_copy(hout_sh.at[sl], buf_v)
        pltpu.sync_copy(buf_v, d00_hbm.at[sl])
        pltpu.sync_copy(hin_sh.at[sl], buf_v)
        pltpu.sync_copy(buf_v, d01_hbm.at[sl])

    @pl.when(c == 1)
    def _():
        pltpu.sync_copy(hout_sh.at[sl], buf_v)
        pltpu.sync_copy(buf_v, d10_hbm.at[sl])
        pltpu.sync_copy(hin_sh.at[sl], buf_v)
        pltpu.sync_copy(buf_v, d11_hbm.at[sl])


GRP = 8
NGRP = NCHUNK // GRP


@functools.partial(
    pl.kernel,
    out_type=jax.ShapeDtypeStruct((NC, NPAD, D), jnp.float32),
    mesh=_mesh,
    scratch_types=[
        pltpu.VMEM((GRP, G), jnp.int32),
        pltpu.VMEM((GRP, G), jnp.int32),
        pltpu.VMEM((G, D), jnp.float32),
        pltpu.VMEM((RCHUNK, D), jnp.float32),
        pltpu.VMEM_SHARED((NPAD, D), jnp.float32),
        pltpu.SemaphoreType.DMA,
    ],
)
def _aggregate_kernel(feat_hbm, src_hbm, dst_hbm, agg_hbm,
                      src_v, dst_v, rows_v, buf_v, agg_sh, sem):
    c = lax.axis_index("c")
    s = lax.axis_index("s")
    w = s * NC + c

    def fill_row(i, carry):
        def fill_col(k, carry2):
            buf_v[i, pl.ds(k * 16, 16)] = jnp.zeros((16,), jnp.float32)
            return carry2

        lax.fori_loop(0, D // 16, fill_col, 0)
        return carry

    lax.fori_loop(0, RCHUNK, fill_row, 0)

    base_row = s * ROWS_PER_TILE
    for r in range(ROWS_PER_TILE // RCHUNK):
        pltpu.sync_copy(buf_v, agg_sh.at[pl.ds(base_row + r * RCHUNK, RCHUNK), :])

    plsc.subcore_barrier()

    def group(g, carry):
        base = w * NCHUNK + g * GRP
        pltpu.sync_copy(src_hbm.at[pl.ds(base, GRP), :], src_v)
        pltpu.sync_copy(dst_hbm.at[pl.ds(base, GRP), :], dst_v)

        def body(j, carry2):
            pltpu.async_copy(feat_hbm.at[src_v.at[j]], rows_v, sem).wait()
            pltpu.sync_copy(rows_v, agg_sh.at[dst_v.at[j]], add=True)
            return carry2

        lax.fori_loop(0, GRP, body, 0)
        return carry

    lax.fori_loop(0, NGRP, group, 0)
    plsc.subcore_barrier()

    sl = pl.ds(base_row, ROWS_PER_TILE)
    pltpu.sync_copy(agg_sh.at[sl, :], agg_hbm.at[c, sl, :])


RB = 1024
NBLK = NPAD // RB


def _feat_body(d00_ref, d10_ref, v_ref, feat_ref):
    d_out = d00_ref[...] + d10_ref[...]
    rs = lax.rsqrt(jnp.maximum(d_out, 1.0))
    feat_ref[...] = v_ref[...] * rs[:, None]


_feat_call = pl.pallas_call(
    _feat_body,
    grid=(NBLK,),
    in_specs=[
        pl.BlockSpec((RB,), lambda i: (i,)),
        pl.BlockSpec((RB,), lambda i: (i,)),
        pl.BlockSpec((RB, D), lambda i: (i, 0)),
    ],
    out_specs=pl.BlockSpec((RB, D), lambda i: (i, 0)),
    out_shape=jax.ShapeDtypeStruct((N, D), jnp.float32),
)


def _out_body(aggp_ref, d01_ref, d11_ref, v_ref, w_ref, b_ref, out_ref):
    agg = aggp_ref[0] + aggp_ref[1]
    d_in = d01_ref[...] + d11_ref[...]
    rs = lax.rsqrt(jnp.maximum(d_in, 1.0))
    rst = agg * rs[:, None]
    rst = jnp.dot(rst, w_ref[...], preferred_element_type=jnp.float32)
    rst = rst + b_ref[...]
    out_ref[...] = jnp.where(rst > 0, rst, SLOPE * rst) + v_ref[...]


_out_call = pl.pallas_call(
    _out_body,
    grid=(NBLK,),
    in_specs=[
        pl.BlockSpec((NC, RB, D), lambda i: (0, i, 0)),
        pl.BlockSpec((RB,), lambda i: (i,)),
        pl.BlockSpec((RB,), lambda i: (i,)),
        pl.BlockSpec((RB, D), lambda i: (i, 0)),
        pl.BlockSpec((D, D), lambda i: (0, 0)),
        pl.BlockSpec((1, D), lambda i: (0, 0)),
    ],
    out_specs=pl.BlockSpec((RB, D), lambda i: (i, 0)),
    out_shape=jax.ShapeDtypeStruct((N, D), jnp.float32),
)


def kernel(V, edge_index, W, b):
    src = edge_index[0].reshape(E // G, G)
    dst = edge_index[1].reshape(E // G, G)
    d00, d01, d10, d11 = _degrees_kernel(src, dst)
    feat = _feat_call(d00, d10, V)
    aggp = _aggregate_kernel(feat, src, dst)
    return _out_call(aggp, d01, d11, V, W, b.reshape(1, D))


# speedup vs baseline: 12.0496x; 1.2699x over previous
import functools

import jax
import jax.numpy as jnp
from jax import lax
from jax.experimental import pallas as pl
from jax.experimental.pallas import tpu as pltpu
from jax.experimental.pallas import tpu_sc as plsc

N = 10000
E = 320000
D = 128
SLOPE = 0.01

NC, NS = 2, 16
NW = NC * NS
G = 125
EPW = E // NW
NCHUNK = EPW // G
NPAD = 10240
DEG_SLICE = NPAD // NS
ROWS_PER_TILE = NPAD // NS
RCHUNK = 64

_mesh = plsc.VectorSubcoreMesh(core_axis_name="c", subcore_axis_name="s")


@functools.partial(
    pl.kernel,
    out_type=tuple(jax.ShapeDtypeStruct((NPAD,), jnp.float32) for _ in range(4)),
    mesh=_mesh,
    scratch_types=[
        pltpu.VMEM((NCHUNK, G), jnp.int32),
        pltpu.VMEM((NCHUNK, G), jnp.int32),
        pltpu.VMEM((128,), jnp.float32),
        pltpu.VMEM((DEG_SLICE,), jnp.float32),
        pltpu.VMEM_SHARED((NPAD,), jnp.float32),
        pltpu.VMEM_SHARED((NPAD,), jnp.float32),
    ],
)
def _degrees_kernel(src_hbm, dst_hbm, d00_hbm, d01_hbm, d10_hbm, d11_hbm,
                    src_v, dst_v, ones_v, buf_v, hout_sh, hin_sh):
    c = lax.axis_index("c")
    s = lax.axis_index("s")
    w = s * NC + c

    def fill_zero(i, carry):
        buf_v[pl.ds(i * 16, 16)] = jnp.zeros((16,), jnp.float32)
        return carry

    lax.fori_loop(0, DEG_SLICE // 16, fill_zero, 0)

    def fill_one(i, carry):
        ones_v[pl.ds(i * 16, 16)] = jnp.ones((16,), jnp.float32)
        return carry

    lax.fori_loop(0, 128 // 16, fill_one, 0)

    sl = pl.ds(s * DEG_SLICE, DEG_SLICE)
    pltpu.sync_copy(buf_v, hout_sh.at[sl])
    pltpu.sync_copy(buf_v, hin_sh.at[sl])
    pltpu.sync_copy(src_hbm.at[pl.ds(w * NCHUNK, NCHUNK), :], src_v)
    pltpu.sync_copy(dst_hbm.at[pl.ds(w * NCHUNK, NCHUNK), :], dst_v)
    plsc.subcore_barrier()

    ones_sl = ones_v.at[pl.ds(0, G)]

    def body(j, carry):
        pltpu.sync_copy(ones_sl, hout_sh.at[src_v.at[j]], add=True)
        pltpu.sync_copy(ones_sl, hin_sh.at[dst_v.at[j]], add=True)
        return carry

    lax.fori_loop(0, NCHUNK, body, 0)
    plsc.subcore_barrier()

    @pl.when(c == 0)
    def _():
        pltpu.sync_copy(hout_sh.at[sl], buf_v)
        pltpu.sync_copy(buf_v, d00_hbm.at[sl])
        pltpu.sync_copy(hin_sh.at[sl], buf_v)
        pltpu.sync_copy(buf_v, d01_hbm.at[sl])

    @pl.when(c == 1)
    def _():
        pltpu.sync_copy(hout_sh.at[sl], buf_v)
        pltpu.sync_copy(buf_v, d10_hbm.at[sl])
        pltpu.sync_copy(hin_sh.at[sl], buf_v)
        pltpu.sync_copy(buf_v, d11_hbm.at[sl])


GRP = 8
NGRP = NCHUNK // GRP


@functools.partial(
    pl.kernel,
    out_type=jax.ShapeDtypeStruct((NC, NPAD, D), jnp.float32),
    mesh=_mesh,
    scratch_types=[
        pltpu.VMEM((NCHUNK, G), jnp.int32),
        pltpu.VMEM((GRP, G), jnp.int32),
        pltpu.VMEM((G, D), jnp.float32),
        pltpu.VMEM((G, D), jnp.float32),
        pltpu.VMEM_SHARED((NPAD, D), jnp.float32),
        pltpu.SemaphoreType.DMA,
        pltpu.SemaphoreType.DMA,
    ],
)
def _aggregate_kernel(feat_hbm, src_hbm, dst_hbm, agg_hbm,
                      src_v, dst_v, rows0_v, rows1_v, agg_sh, sem0, sem1):
    c = lax.axis_index("c")
    s = lax.axis_index("s")
    w = s * NC + c

    def fill_row(i, carry):
        def fill_col(k, carry2):
            rows0_v[i, pl.ds(k * 16, 16)] = jnp.zeros((16,), jnp.float32)
            return carry2

        lax.fori_loop(0, D // 16, fill_col, 0)
        return carry

    lax.fori_loop(0, G, fill_row, 0)

    base_row = s * ROWS_PER_TILE
    for r in range(ROWS_PER_TILE // G):
        pltpu.sync_copy(rows0_v, agg_sh.at[pl.ds(base_row + r * G, G), :])
    pltpu.sync_copy(rows0_v.at[pl.ds(0, ROWS_PER_TILE % G), :],
                    agg_sh.at[pl.ds(base_row + (ROWS_PER_TILE // G) * G,
                                    ROWS_PER_TILE % G), :])

    pltpu.sync_copy(src_hbm.at[pl.ds(w * NCHUNK, NCHUNK), :], src_v)
    pltpu.sync_copy(dst_hbm.at[pl.ds(w * NCHUNK, GRP), :], dst_v)
    plsc.subcore_barrier()

    rows = (rows0_v, rows1_v)
    sems = (sem0, sem1)
    pltpu.async_copy(feat_hbm.at[src_v.at[0]], rows0_v, sem0)

    def body(i, carry):
        t2 = i * 2
        for b in range(2):
            t = t2 + b
            cur, nxt = rows[b], rows[1 - b]
            scur, snxt = sems[b], sems[1 - b]
            if b == 0:
                @pl.when(jnp.logical_and(t % GRP == 0, t > 0))
                def _():
                    pltpu.sync_copy(
                        dst_hbm.at[pl.ds(w * NCHUNK + (t // GRP) * GRP, GRP), :],
                        dst_v)
            pltpu.make_async_copy(feat_hbm.at[src_v.at[t]], cur, scur).wait()

            @pl.when(t + 1 < NCHUNK)
            def _():
                pltpu.async_copy(feat_hbm.at[src_v.at[t + 1]], nxt, snxt)

            pltpu.sync_copy(cur, agg_sh.at[dst_v.at[t % GRP]], add=True)
        return carry

    lax.fori_loop(0, NCHUNK // 2, body, 0)
    plsc.subcore_barrier()

    sl = pl.ds(base_row, ROWS_PER_TILE)
    pltpu.sync_copy(agg_sh.at[sl, :], agg_hbm.at[c, sl, :])


RB = 1024
NBLK = NPAD // RB


def _feat_body(d00_ref, d10_ref, v_ref, feat_ref):
    d_out = d00_ref[...] + d10_ref[...]
    rs = lax.rsqrt(jnp.maximum(d_out, 1.0))
    feat_ref[...] = v_ref[...] * rs[:, None]


_feat_call = pl.pallas_call(
    _feat_body,
    grid=(NBLK,),
    in_specs=[
        pl.BlockSpec((RB,), lambda i: (i,)),
        pl.BlockSpec((RB,), lambda i: (i,)),
        pl.BlockSpec((RB, D), lambda i: (i, 0)),
    ],
    out_specs=pl.BlockSpec((RB, D), lambda i: (i, 0)),
    out_shape=jax.ShapeDtypeStruct((N, D), jnp.float32),
)


def _out_body(aggp_ref, d01_ref, d11_ref, v_ref, w_ref, b_ref, out_ref):
    agg = aggp_ref[0] + aggp_ref[1]
    d_in = d01_ref[...] + d11_ref[...]
    rs = lax.rsqrt(jnp.maximum(d_in, 1.0))
    rst = agg * rs[:, None]
    rst = jnp.dot(rst, w_ref[...], preferred_element_type=jnp.float32)
    rst = rst + b_ref[...]
    out_ref[...] = jnp.where(rst > 0, rst, SLOPE * rst) + v_ref[...]


_out_call = pl.pallas_call(
    _out_body,
    grid=(NBLK,),
    in_specs=[
        pl.BlockSpec((NC, RB, D), lambda i: (0, i, 0)),
        pl.BlockSpec((RB,), lambda i: (i,)),
        pl.BlockSpec((RB,), lambda i: (i,)),
        pl.BlockSpec((RB, D), lambda i: (i, 0)),
        pl.BlockSpec((D, D), lambda i: (0, 0)),
        pl.BlockSpec((1, D), lambda i: (0, 0)),
    ],
    out_specs=pl.BlockSpec((RB, D), lambda i: (i, 0)),
    out_shape=jax.ShapeDtypeStruct((N, D), jnp.float32),
)


def kernel(V, edge_index, W, b):
    src = edge_index[0].reshape(E // G, G)
    dst = edge_index[1].reshape(E // G, G)
    d00, d01, d10, d11 = _degrees_kernel(src, dst)
    feat = _feat_call(d00, d10, V)
    aggp = _aggregate_kernel(feat, src, dst)
    return _out_call(aggp, d01, d11, V, W, b.reshape(1, D))
